# baseline (speedup 1.0000x reference)
import functools

import jax
import jax.numpy as jnp
from jax import lax
from jax.experimental import pallas as pl
from jax.experimental.pallas import tpu as pltpu
from jax.experimental.pallas import tpu_sc as plsc

_GRIDS = (13, 26, 52)
_A = 3
_NGT = 10
_NC = 80
_THR = 0.5
_WHS = (
    ((3.625, 2.8125), (4.875, 6.1875), (11.65625, 10.1875)),
    ((1.875, 3.8125), (3.875, 2.8125), (3.6875, 7.4375)),
    ((1.25, 1.625), (2.0, 3.75), (4.125, 2.875)),
)
_B = 8


def _sc_assign(gt_hbm, tgt_o, gt_v, tgt_v, s_idx, b, grid, whs):
    pltpu.sync_copy(gt_hbm.at[b], gt_v)

    def col(j):
        return gt_v[j, :]

    x1 = col(0) * grid
    y1 = col(1) * grid
    x2 = col(2) * grid
    y2 = col(3) * grid
    clsf = col(4)
    cx = (x1 + x2) * 0.5
    cy = (y1 + y2) * 0.5
    w = x2 - x1
    h = y2 - y1
    area = w * h
    r_i = cy.astype(jnp.int32)
    c_i = cx.astype(jnp.int32)
    rf = r_i.astype(jnp.float32)
    cf = c_i.astype(jnp.float32)
    acx = cf + 0.5
    acy = rf + 0.5
    best_a = jnp.zeros((16,), jnp.float32)
    best_v = jnp.full((16,), -1.0, jnp.float32)
    for a, (wa, ha) in enumerate(whs):
        ix = jnp.maximum(
            jnp.minimum(acx + wa * 0.5, x2) - jnp.maximum(acx - wa * 0.5, x1),
            0.0)
        iy = jnp.maximum(
            jnp.minimum(acy + ha * 0.5, y2) - jnp.maximum(acy - ha * 0.5, y1),
            0.0)
        inter = ix * iy
        union = jnp.maximum(wa * ha + area - inter, 1e-16)
        iou = inter / union
        take = iou > best_v
        best_a = jnp.where(take, jnp.float32(a), best_a)
        best_v = jnp.where(take, iou, best_v)
    hitf = jnp.where(best_v >= _THR, 1.0, 0.0)
    nf = (rf * grid + cf) * 3.0 + best_a

    tgt_v[0, :] = hitf
    tgt_v[1, :] = best_a
    tgt_v[2, :] = cx - cf
    tgt_v[3, :] = cy - rf
    tgt_v[4, :] = w
    tgt_v[5, :] = h
    tgt_v[6, :] = 2.0 - area / float(grid * grid)
    tgt_v[7, :] = clsf
    tgt_v[8, :] = nf
    tgt_v[9, :] = rf
    tgt_v[10, :] = cf
    tgt_v[11, :] = jnp.zeros((16,), jnp.float32)
    pltpu.sync_copy(tgt_v, tgt_o.at[s_idx, b])


def _sc_body(gt_hbm, tgt_o, gt_v, tgt_v):
    wid = lax.axis_index("s") * 2 + lax.axis_index("c")
    for s_idx in range(3):
        lo = 8 * s_idx

        @pl.when((wid >= lo) & (wid < lo + 8))
        def _(s_idx=s_idx, lo=lo):
            _sc_assign(gt_hbm, tgt_o, gt_v, tgt_v,
                       s_idx, wid - lo, _GRIDS[s_idx], _WHS[s_idx])


def _sc_call(gt_t):
    mesh = plsc.VectorSubcoreMesh(core_axis_name="c", subcore_axis_name="s")
    fn = functools.partial(
        pl.kernel, mesh=mesh,
        out_type=jax.ShapeDtypeStruct((3, _B, 12, 16), jnp.float32),
        scratch_types=[
            pltpu.VMEM((5, 16), jnp.float32),
            pltpu.VMEM((12, 16), jnp.float32),
        ],
    )(_sc_body)
    return fn(gt_t)


def _main_kernel(gt_ref, tgt_s_ref, tgt_ref, p0_ref, p1_ref, p2_ref,
                 out_ref, rows_sc):
    b = pl.program_id(0)
    gts = []
    for gi in range(_NGT):
        gts.append(tuple(gt_ref[0, gi, j] for j in range(5)))
    p_refs = (p0_ref, p1_ref, p2_ref)
    total = jnp.float32(0.0)

    for s_idx in range(3):
        for gi in range(_NGT):
            r = tgt_s_ref[s_idx, 0, 9, gi].astype(jnp.int32)
            c = tgt_s_ref[s_idx, 0, 10, gi].astype(jnp.int32)
            v = p_refs[s_idx][0, pl.ds(r, 1), pl.ds(c, 1), :]
            rows_sc[gi + 16 * s_idx:gi + 16 * s_idx + 1, :] = \
                jnp.reshape(v, (1, 255))

    for s_idx in range(3):
        grid = _GRIDS[s_idx]
        whs = _WHS[s_idx]
        geo = []
        for (x1, y1, x2, y2, cfv) in gts:
            gx1, gy1, gx2, gy2 = x1 * grid, y1 * grid, x2 * grid, y2 * grid
            area = (gx2 - gx1) * (gy2 - gy1)
            geo.append((gx1, gy1, gx2, gy2, area))
        rows = jax.lax.broadcasted_iota(jnp.int32, (grid, grid), 0).astype(
            jnp.float32)
        cols = jax.lax.broadcasted_iota(jnp.int32, (grid, grid), 1).astype(
            jnp.float32)
        for a, (wa, ha) in enumerate(whs):
            ax1 = cols + (0.5 - wa * 0.5)
            ax2 = cols + (0.5 + wa * 0.5)
            ay1 = rows + (0.5 - ha * 0.5)
            ay2 = rows + (0.5 + ha * 0.5)
            area_a = wa * ha
            ges = []
            for (gx1, gy1, gx2, gy2, area) in geo:
                ix = jnp.maximum(
                    jnp.minimum(ax2, gx2) - jnp.maximum(ax1, gx1), 0.0)
                iy = jnp.maximum(
                    jnp.minimum(ay2, gy2) - jnp.maximum(ay1, gy1), 0.0)
                inter = ix * iy
                union = jnp.maximum(area_a + area - inter, 1e-16)
                ges.append((inter / union) >= _THR)
            while len(ges) > 1:
                ges = [a_ | b_ for a_, b_ in zip(ges[::2], ges[1::2])] + (
                    [ges[-1]] if len(ges) % 2 else [])
            zc = p_refs[s_idx][0, :, :, 85 * a + 4]
            l1p = jnp.maximum(jnp.log1p(-jax.nn.sigmoid(zc)), -100.0)
            total = total - jnp.sum(jnp.where(ges[0], 0.0, l1p))

    sub16 = jax.lax.broadcasted_iota(jnp.int32, (16, 16), 0)
    lan16 = jax.lax.broadcasted_iota(jnp.int32, (16, 16), 1)
    lane80 = jax.lax.broadcasted_iota(jnp.int32, (16, _NC), 1)
    for s_idx in range(3):
        whs = _WHS[s_idx]
        traw = tgt_ref[s_idx, 0]
        t = jnp.transpose(traw)
        hit_s = t[:, 0:1] > 0.5
        af = t[:, 1:2]
        tx = t[:, 2:3]
        ty = t[:, 3:4]
        w = t[:, 4:5]
        h = t[:, 5:6]
        gs = t[:, 6:7]
        cls16 = (t[:, 7:8] - 1.0).astype(jnp.int32)
        n_s = t[:, 8:9]
        hit_l = traw[0:1, :] > 0.5
        n_l = traw[8:9, :]
        clobm = jnp.where((n_s == n_l) & hit_l & (sub16 < lan16) &
                          (lan16 < _NGT), 1.0, 0.0)
        live = hit_s & (jnp.max(clobm, axis=1, keepdims=True) < 0.5)
        rows = rows_sc[16 * s_idx:16 * s_idx + 16, :]
        for a, (wa, ha) in enumerate(whs):
            base = 85 * a
            zx = rows[:, base + 0:base + 1]
            zy = rows[:, base + 1:base + 2]
            zw = rows[:, base + 2:base + 3]
            zh = rows[:, base + 3:base + 4]
            zc = rows[:, base + 4:base + 5]
            logits = rows[:, base + 5:base + 85]
            m = jnp.max(logits, axis=1, keepdims=True)
            lse = m + jnp.log(jnp.sum(jnp.exp(logits - m), axis=1,
                                      keepdims=True))
            sel = jnp.sum(jnp.where(lane80 == cls16, logits, 0.0),
                          axis=1, keepdims=True)
            tw = jnp.log(w / wa + 1e-16)
            th = jnp.log(h / ha + 1e-16)
            lxy = ((jax.nn.sigmoid(zx) - tx) ** 2
                   + (jax.nn.sigmoid(zy) - ty) ** 2)
            lwh = (jnp.tanh(zw) - tw) ** 2 + (jnp.tanh(zh) - th) ** 2
            lco = -jnp.maximum(jnp.log(jax.nn.sigmoid(zc)), -100.0)
            term = gs * (lxy + lwh) + (lse - sel) + lco
            mask = live & (af == float(a))
            total = total + jnp.sum(jnp.where(mask, term, 0.0))

    @pl.when(b == 0)
    def _init():
        out_ref[0] = jnp.float32(0.0)

    out_ref[0] += total


def _main_call(gt, tgt, pred0, pred1, pred2):
    return pl.pallas_call(
        _main_kernel,
        grid=(_B,),
        in_specs=[
            pl.BlockSpec((1, _NGT, 5), lambda b: (b, 0, 0),
                         memory_space=pltpu.SMEM),
            pl.BlockSpec((3, 1, 12, 16), lambda b: (0, b, 0, 0),
                         memory_space=pltpu.SMEM),
            pl.BlockSpec((3, 1, 12, 16), lambda b: (0, b, 0, 0)),
            pl.BlockSpec((1, 13, 13, 255), lambda b: (b, 0, 0, 0)),
            pl.BlockSpec((1, 26, 26, 255), lambda b: (b, 0, 0, 0)),
            pl.BlockSpec((1, 52, 52, 255), lambda b: (b, 0, 0, 0)),
        ],
        out_specs=pl.BlockSpec((1,), lambda b: (0,),
                               memory_space=pltpu.SMEM),
        out_shape=jax.ShapeDtypeStruct((1,), jnp.float32),
        scratch_shapes=[pltpu.VMEM((48, 255), jnp.float32)],
    )(gt, tgt, tgt, pred0, pred1, pred2)


@jax.jit
def kernel(pred0, pred1, pred2, gt_bbox):
    gt_t = jnp.pad(jnp.swapaxes(gt_bbox, 1, 2), ((0, 0), (0, 0), (0, 6)))
    tgt = _sc_call(gt_t)
    return _main_call(gt_bbox, tgt, pred0, pred1, pred2)


# speedup vs baseline: 1.2675x; 1.2675x over previous
import functools

import jax
import jax.numpy as jnp
from jax import lax
from jax.experimental import pallas as pl
from jax.experimental.pallas import tpu as pltpu
from jax.experimental.pallas import tpu_sc as plsc

_GRIDS = (13, 26, 52)
_A = 3
_NGT = 10
_NC = 80
_THR = 0.5
_WHS = (
    ((3.625, 2.8125), (4.875, 6.1875), (11.65625, 10.1875)),
    ((1.875, 3.8125), (3.875, 2.8125), (3.6875, 7.4375)),
    ((1.25, 1.625), (2.0, 3.75), (4.125, 2.875)),
)
_B = 8
_CONF_BLK = (0, 11, 21)
_CONF_OFF = (4, 1, 6)


def _sc_assign(gt_hbm, tgt_o, gt_v, tgt_v, s_idx, b, grid, whs):
    pltpu.sync_copy(gt_hbm.at[b], gt_v)

    def col(j):
        return gt_v[j, :]

    x1 = col(0) * grid
    y1 = col(1) * grid
    x2 = col(2) * grid
    y2 = col(3) * grid
    clsf = col(4)
    cx = (x1 + x2) * 0.5
    cy = (y1 + y2) * 0.5
    w = x2 - x1
    h = y2 - y1
    area = w * h
    r_i = cy.astype(jnp.int32)
    c_i = cx.astype(jnp.int32)
    rf = r_i.astype(jnp.float32)
    cf = c_i.astype(jnp.float32)
    acx = cf + 0.5
    acy = rf + 0.5
    best_a = jnp.zeros((16,), jnp.float32)
    best_v = jnp.full((16,), -1.0, jnp.float32)
    for a, (wa, ha) in enumerate(whs):
        ix = jnp.maximum(
            jnp.minimum(acx + wa * 0.5, x2) - jnp.maximum(acx - wa * 0.5, x1),
            0.0)
        iy = jnp.maximum(
            jnp.minimum(acy + ha * 0.5, y2) - jnp.maximum(acy - ha * 0.5, y1),
            0.0)
        inter = ix * iy
        union = jnp.maximum(wa * ha + area - inter, 1e-16)
        iou = inter / union
        take = iou > best_v
        best_a = jnp.where(take, jnp.float32(a), best_a)
        best_v = jnp.where(take, iou, best_v)
    hitf = jnp.where(best_v >= _THR, 1.0, 0.0)
    nf = (rf * grid + cf) * 3.0 + best_a

    tgt_v[0, :] = hitf
    tgt_v[1, :] = best_a
    tgt_v[2, :] = cx - cf
    tgt_v[3, :] = cy - rf
    tgt_v[4, :] = w
    tgt_v[5, :] = h
    tgt_v[6, :] = 2.0 - area / float(grid * grid)
    tgt_v[7, :] = clsf
    tgt_v[8, :] = nf
    tgt_v[9, :] = rf
    tgt_v[10, :] = cf
    tgt_v[11, :] = jnp.zeros((16,), jnp.float32)
    pltpu.sync_copy(tgt_v, tgt_o.at[s_idx, b])


def _sc_body(gt_hbm, tgt_o, gt_v, tgt_v):
    wid = lax.axis_index("s") * 2 + lax.axis_index("c")
    for s_idx in range(3):
        lo = 8 * s_idx

        @pl.when((wid >= lo) & (wid < lo + 8))
        def _(s_idx=s_idx, lo=lo):
            _sc_assign(gt_hbm, tgt_o, gt_v, tgt_v,
                       s_idx, wid - lo, _GRIDS[s_idx], _WHS[s_idx])


def _sc_call(gt_t):
    mesh = plsc.VectorSubcoreMesh(core_axis_name="c", subcore_axis_name="s")
    fn = functools.partial(
        pl.kernel, mesh=mesh,
        out_type=jax.ShapeDtypeStruct((3, _B, 12, 16), jnp.float32),
        scratch_types=[
            pltpu.VMEM((5, 16), jnp.float32),
            pltpu.VMEM((12, 16), jnp.float32),
        ],
    )(_sc_body)
    return fn(gt_t)


def _dense_kernel(gt_ref, tgt_s_ref, tgt_ref, p0_ref, p1_ref, p2_ref,
                  ph0_ref, ph1_ref, ph2_ref, out_ref, rows_v, sem):
    b = pl.program_id(0)
    hb_preds = (ph0_ref, ph1_ref, ph2_ref)

    @pl.when(b == 0)
    def _issue():
        _row_dmas(tgt_s_ref, hb_preds, rows_v, sem, 0)
    gts = []
    for gi in range(_NGT):
        gts.append(tuple(gt_ref[0, gi, j] for j in range(5)))
    p_refs = (p0_ref, p1_ref, p2_ref)
    total = jnp.float32(0.0)
    for s_idx in range(3):
        grid = _GRIDS[s_idx]
        whs = _WHS[s_idx]
        geo = []
        for (x1, y1, x2, y2, cfv) in gts:
            gx1, gy1, gx2, gy2 = x1 * grid, y1 * grid, x2 * grid, y2 * grid
            area = (gx2 - gx1) * (gy2 - gy1)
            geo.append((gx1, gy1, gx2, gy2, area))
        rows = jax.lax.broadcasted_iota(jnp.int32, (grid, grid), 0).astype(
            jnp.float32)
        cols = jax.lax.broadcasted_iota(jnp.int32, (grid, grid), 1).astype(
            jnp.float32)
        for a, (wa, ha) in enumerate(whs):
            ax1 = cols + (0.5 - wa * 0.5)
            ax2 = cols + (0.5 + wa * 0.5)
            ay1 = rows + (0.5 - ha * 0.5)
            ay2 = rows + (0.5 + ha * 0.5)
            area_a = wa * ha
            ges = []
            for (gx1, gy1, gx2, gy2, area) in geo:
                ix = jnp.maximum(
                    jnp.minimum(ax2, gx2) - jnp.maximum(ax1, gx1), 0.0)
                iy = jnp.maximum(
                    jnp.minimum(ay2, gy2) - jnp.maximum(ay1, gy1), 0.0)
                inter = ix * iy
                union = jnp.maximum(area_a + area - inter, 1e-16)
                ges.append((inter / union) >= _THR)
            while len(ges) > 1:
                ges = [a_ | b_ for a_, b_ in zip(ges[::2], ges[1::2])] + (
                    [ges[-1]] if len(ges) % 2 else [])
            zc = p_refs[s_idx][0, :, :, 85 * a + 4]
            l1p = jnp.maximum(jnp.log1p(-jax.nn.sigmoid(zc)), -100.0)
            total = total - jnp.sum(jnp.where(ges[0], 0.0, l1p))

    @pl.when(b == 0)
    def _init():
        out_ref[0] = jnp.float32(0.0)

    out_ref[0] += total

    @pl.when(b == _B - 1)
    def _sparse():
        _row_dmas(tgt_s_ref, hb_preds, rows_v, sem, 1)
        out_ref[0] += _sparse_terms(tgt_ref, rows_v)


def _row_dmas(tgt_s_ref, preds, rows_v, sem, phase):
    for s_idx in range(3):
        for bb in range(_B):
            for gi in range(_NGT):
                hitv = tgt_s_ref[s_idx, bb, 0, gi] > 0.5
                r = tgt_s_ref[s_idx, bb, 9, gi].astype(jnp.int32)
                c = tgt_s_ref[s_idx, bb, 10, gi].astype(jnp.int32)
                dst = (s_idx * _B + bb) * 16 + gi
                cp = pltpu.make_async_copy(
                    preds[s_idx].at[bb, pl.ds(r, 1), pl.ds(c, 1), :],
                    rows_v.at[pl.ds(0, 1), pl.ds(dst, 1), :], sem)

                @pl.when(hitv)
                def _(cp=cp, phase=phase):
                    if phase == 0:
                        cp.start()
                    else:
                        cp.wait()


def _sparse_terms(tgt_ref, rows_v):
    NR = 3 * _B * 16
    sub16 = jax.lax.broadcasted_iota(jnp.int32, (16, 16), 0)
    lan16 = jax.lax.broadcasted_iota(jnp.int32, (16, 16), 1)
    t_parts = []
    live_parts = []
    for s_idx in range(3):
        for bb in range(_B):
            traw = tgt_ref[s_idx, bb]
            t = jnp.transpose(traw)
            hit_s = t[:, 0:1] > 0.5
            n_s = t[:, 8:9]
            hit_l = traw[0:1, :] > 0.5
            n_l = traw[8:9, :]
            clobm = jnp.where((n_s == n_l) & hit_l & (sub16 < lan16) &
                              (lan16 < _NGT), 1.0, 0.0)
            live = hit_s & (jnp.max(clobm, axis=1, keepdims=True) < 0.5)
            t_parts.append(t)
            live_parts.append(jnp.where(live, 1.0, 0.0))
    t_all = jnp.concatenate(t_parts, axis=0)
    live_all = jnp.concatenate(live_parts, axis=0)

    af = t_all[:, 1:2]
    tx = t_all[:, 2:3]
    ty = t_all[:, 3:4]
    w = t_all[:, 4:5]
    h = t_all[:, 5:6]
    gs = t_all[:, 6:7]
    cls_i = (t_all[:, 7:8] - 1.0).astype(jnp.int32)
    srow = jax.lax.broadcasted_iota(jnp.int32, (NR, 1), 0) // (_B * 16)
    wa_all = jnp.zeros((NR, 1), jnp.float32)
    ha_all = jnp.zeros((NR, 1), jnp.float32)
    for s_idx in range(3):
        for a in range(3):
            m_sa = (srow == s_idx) & (af == float(a))
            wa_all = jnp.where(m_sa, _WHS[s_idx][a][0], wa_all)
            ha_all = jnp.where(m_sa, _WHS[s_idx][a][1], ha_all)

    rows = rows_v[0]
    lane80 = jax.lax.broadcasted_iota(jnp.int32, (NR, _NC), 1)
    acc = jnp.zeros((NR, 1), jnp.float32)
    for a in range(3):
        base = 85 * a
        zx = rows[:, base + 0:base + 1]
        zy = rows[:, base + 1:base + 2]
        zw = rows[:, base + 2:base + 3]
        zh = rows[:, base + 3:base + 4]
        zc = rows[:, base + 4:base + 5]
        logits = rows[:, base + 5:base + 85]
        m = jnp.max(logits, axis=1, keepdims=True)
        lse = m + jnp.log(jnp.sum(jnp.exp(logits - m), axis=1,
                                  keepdims=True))
        sel = jnp.sum(jnp.where(lane80 == cls_i, logits, 0.0),
                      axis=1, keepdims=True)
        tw = jnp.log(w / wa_all + 1e-16)
        th = jnp.log(h / ha_all + 1e-16)
        lxy = ((jax.nn.sigmoid(zx) - tx) ** 2
               + (jax.nn.sigmoid(zy) - ty) ** 2)
        lwh = (jnp.tanh(zw) - tw) ** 2 + (jnp.tanh(zh) - th) ** 2
        lco = -jnp.maximum(jnp.log(jax.nn.sigmoid(zc)), -100.0)
        term = gs * (lxy + lwh) + (lse - sel) + lco
        mask = (live_all > 0.5) & (af == float(a))
        acc = acc + jnp.where(mask, term, 0.0)
    return jnp.sum(acc)


def _dense_call(gt, tgt, pred0, pred1, pred2):
    in_specs = [
        pl.BlockSpec((1, _NGT, 5), lambda b: (b, 0, 0),
                     memory_space=pltpu.SMEM),
        pl.BlockSpec((3, _B, 12, 16), lambda b: (0, 0, 0, 0),
                     memory_space=pltpu.SMEM),
        pl.BlockSpec((3, _B, 12, 16), lambda b: (0, 0, 0, 0)),
        pl.BlockSpec((1, 13, 13, 255), lambda b: (b, 0, 0, 0)),
        pl.BlockSpec((1, 26, 26, 255), lambda b: (b, 0, 0, 0)),
        pl.BlockSpec((1, 52, 52, 255), lambda b: (b, 0, 0, 0)),
        pl.BlockSpec(memory_space=pl.ANY),
        pl.BlockSpec(memory_space=pl.ANY),
        pl.BlockSpec(memory_space=pl.ANY),
    ]
    args = [gt, tgt, tgt, pred0, pred1, pred2, pred0, pred1, pred2]
    return pl.pallas_call(
        _dense_kernel,
        grid=(_B,),
        in_specs=in_specs,
        out_specs=pl.BlockSpec((1,), lambda b: (0,),
                               memory_space=pltpu.SMEM),
        out_shape=jax.ShapeDtypeStruct((1,), jnp.float32),
        scratch_shapes=[
            pltpu.VMEM((1, 3 * _B * 16, 255), jnp.float32),
            pltpu.SemaphoreType.DMA,
        ],
    )(*args)


@jax.jit
def kernel(pred0, pred1, pred2, gt_bbox):
    gt_t = jnp.pad(jnp.swapaxes(gt_bbox, 1, 2), ((0, 0), (0, 0), (0, 6)))
    tgt = _sc_call(gt_t)
    return _dense_call(gt_bbox, tgt, pred0, pred1, pred2)
